# Initial kernel scaffold; baseline (speedup 1.0000x reference)
import jax
import jax.numpy as jnp
from jax.experimental import pallas as pl


def kernel(features, ks, imu2cs, post_rots, post_trans, undists, grid, drop_idx, neck):
    raise NotImplementedError("write your pallas kernel here")


# speedup vs baseline: 2.6201x; 2.6201x over previous
import functools

import jax
import jax.numpy as jnp
from jax import lax
from jax.experimental import pallas as pl
from jax.experimental.pallas import tpu as pltpu
from jax.experimental.pallas import tpu_sc as plsc

B, N, C, H, W = 2, 6, 64, 112, 200
HC, DG, WG = 7, 200, 200
P = HC * DG * WG
RPI = H * W
ZROW = B * N * RPI
MROW = ZROW + 1
ZOFFS = [2.0, 1.5, 1.0, 0.5, 0.0, -0.5, -1.0]

NCORE, NSUB, LANES = 2, 16, 16
NW = NCORE * NSUB
K = 112
CHUNKS_PER_B = P // K
TOTAL_CHUNKS = B * CHUNKS_PER_B
NT = -(-TOTAL_CHUNKS // NW)


def _atan_pos(r):
    big = r > 2.414213562373095
    mid = r > 0.4142135623730951
    x = jnp.where(big, -1.0 / r, jnp.where(mid, (r - 1.0) / (r + 1.0), r))
    y = jnp.where(
        big,
        jnp.float32(1.5707963267948966),
        jnp.where(mid, jnp.float32(0.7853981633974483), jnp.float32(0.0)),
    )
    z = x * x
    p = (
        ((8.05374449538e-2 * z - 1.38776856032e-1) * z + 1.99777106478e-1) * z
        - 3.33329491539e-1
    ) * z * x + x
    return y + p


def _project_body(gx_ref, gy_ref, gz_ref, fp_ref, ip_ref, idx_ref):
    b = pl.program_id(0)
    i = pl.program_id(1)

    def fp(j):
        return fp_ref[b, i, j]

    c00, c01, c02, c03 = fp(0), fp(1), fp(2), fp(3)
    c10, c11, c12, c13 = fp(4), fp(5), fp(6), fp(7)
    c20, c21, c22, c23 = fp(8), fp(9), fp(10), fp(11)
    fx, fy, cx, cy = fp(12), fp(13), fp(14), fp(15)
    d0, d1, d2, d5 = fp(16), fp(17), fp(18), fp(19)
    k1, k2, k3, p1, p2 = fp(20), fp(21), fp(22), fp(23), fp(24)
    sel_fe = fp(25) > 0.5
    pr00, pr01, pr10, pr11 = fp(26), fp(27), fp(28), fp(29)
    pt0, pt1 = fp(30), fp(31)
    base = ip_ref[b, i, 0]
    dropped = ip_ref[b, i, 1] == 1

    gx = gx_ref[0]
    gy = gy_ref[0]
    gz0 = gz_ref[0]

    def b16(v):
        return v.astype(jnp.bfloat16).astype(jnp.float32)

    gxb = b16(gx)
    gyb = b16(gy)

    for h in range(HC):
        gz = gz0 + ZOFFS[h]
        gzb = b16(gz)
        hx = (b16(c00) * gxb + b16(c01) * gyb) + b16(c02) * gzb + c03
        hy = (b16(c10) * gxb + b16(c11) * gyb) + b16(c12) * gzb + c13
        hz = (b16(c20) * gxb + b16(c21) * gyb) + b16(c22) * gzb + c23
        f = (hz > 0.0).astype(jnp.float32)
        px = (hx * f) / hz
        py = (hy * f) / hz

        x = (px - cx) / fx
        y = (py - cy) / fy

        r = jnp.sqrt(x * x + y * y)
        th = _atan_pos(r)
        t2 = th * th
        t4 = t2 * t2
        rad = th * (1.0 + d0 * t2 + d1 * t4 + d2 * t4 * t2 + d5 * t4 * t4) / r
        fex = ((x * rad) * fx + cx) * f
        fey = ((y * rad) * fy + cy) * f

        r2 = x * x + y * y
        radial = 1.0 + k1 * r2 + k2 * r2 * r2 + k3 * r2 * r2 * r2
        phx = (x * radial + (2.0 * p1 * x * y + p2 * (r2 + 2.0 * x * x))) * fx + cx
        phy = (y * radial + (p1 * (r2 + 2.0 * y * y) + 2.0 * p2 * x * y)) * fy + cy
        phx = phx * f
        phy = phy * f

        xd = b16(jnp.where(sel_fe, fex, phx))
        yd = b16(jnp.where(sel_fe, fey, phy))

        u = (b16(pr00) * xd + b16(pr01) * yd) + pt0
        v = (b16(pr10) * xd + b16(pr11) * yd) + pt1

        bx = u.astype(jnp.int32)
        by = v.astype(jnp.int32)
        vis = (bx > 0) & (bx < W) & (by < H) & (by > 0)
        xx = jnp.clip(bx, 0, W - 1)
        yy = jnp.clip(by, 0, H - 1)
        row = base + yy * W + xx
        idx = jnp.where(vis, row, ZROW)
        idx = jnp.where(dropped, MROW, idx)
        idx_ref[0, 0, h] = idx


_project = pl.pallas_call(
    _project_body,
    grid=(B, N),
    in_specs=[
        pl.BlockSpec((1, DG, WG), lambda b, i: (b, 0, 0)),
        pl.BlockSpec((1, DG, WG), lambda b, i: (b, 0, 0)),
        pl.BlockSpec((1, DG, WG), lambda b, i: (b, 0, 0)),
        pl.BlockSpec(memory_space=pltpu.SMEM),
        pl.BlockSpec(memory_space=pltpu.SMEM),
    ],
    out_specs=pl.BlockSpec((1, 1, HC, DG, WG), lambda b, i: (b, i, 0, 0, 0)),
    out_shape=jax.ShapeDtypeStruct((B, N, HC, DG, WG), jnp.int32),
)


def _gather_max_body(table_hbm, idx_hbm, out_hbm, idx_v, rows_v, tbuf, gsem):
    wid = lax.axis_index("s") * NCORE + lax.axis_index("c")
    iota = lax.iota(jnp.int32, LANES)

    def step(t, carry):
        cid = wid + t * NW

        @pl.when(cid < TOTAL_CHUNKS)
        def _():
            b = cid // CHUNKS_PER_B
            pb = (cid % CHUNKS_PER_B) * K
            pltpu.sync_copy(idx_hbm.at[b, :, pl.ds(pb, K)], idx_v)
            descs = [
                pltpu.async_copy(table_hbm.at[idx_v.at[i]], rows_v.at[i], gsem)
                for i in range(N)
            ]
            for d in descs:
                d.wait()

            def combine(j, carry2):
                jvec = jnp.zeros((LANES,), jnp.int32) + j
                for c4 in range(C // LANES):
                    s = pl.ds(c4 * LANES, LANES)
                    m = rows_v[0, j, s]
                    for i in range(1, N):
                        m = jnp.maximum(m, rows_v[i, j, s])
                    plsc.store_scatter(tbuf, [iota + c4 * LANES, jvec], m)
                return carry2

            lax.fori_loop(0, K, combine, 0)
            pltpu.sync_copy(tbuf, out_hbm.at[b, :, pl.ds(pb, K)])

        return carry

    lax.fori_loop(0, NT, step, 0)


@functools.lru_cache(maxsize=1)
def _gather_max():
    return pl.kernel(
        _gather_max_body,
        out_type=jax.ShapeDtypeStruct((B, C, P), jnp.float32),
        mesh=plsc.VectorSubcoreMesh(
            core_axis_name="c", subcore_axis_name="s",
            num_cores=NCORE, num_subcores=NSUB,
        ),
        scratch_types=[
            pltpu.VMEM((N, K), jnp.int32),
            pltpu.VMEM((N, K, C), jnp.float32),
            pltpu.VMEM((C, K), jnp.float32),
            pltpu.SemaphoreType.DMA,
        ],
        compiler_params=pltpu.CompilerParams(
            use_tc_tiling_on_sc=False, needs_layout_passes=False
        ),
    )


@jax.jit
def kernel(features, ks, imu2cs, post_rots, post_trans, undists, grid, drop_idx, neck):
    f32 = jnp.float32
    calib = jnp.matmul(ks, imu2cs)

    fparams = jnp.concatenate(
        [
            calib.reshape(B, N, 12).astype(f32),
            jnp.stack(
                [ks[..., 0, 0], ks[..., 1, 1], ks[..., 0, 2], ks[..., 1, 2]], axis=-1
            ).astype(f32),
            jnp.stack(
                [undists[..., 0], undists[..., 1], undists[..., 2], undists[..., 5]],
                axis=-1,
            ).astype(f32),
            undists[..., 0:5].astype(f32),
            (undists[..., 6] == 1.0).astype(f32)[..., None],
            post_rots[..., :2, :2].reshape(B, N, 4).astype(f32),
            post_trans[..., :2].astype(f32),
        ],
        axis=-1,
    )

    base = ((jnp.arange(B, dtype=jnp.int32)[:, None] * N)
            + jnp.arange(N, dtype=jnp.int32)[None, :]) * RPI
    dropped = jnp.any(
        drop_idx[None, :] == jnp.arange(N, dtype=drop_idx.dtype)[:, None], axis=1
    )
    iparams = jnp.stack(
        [base, jnp.broadcast_to(dropped[None, :], (B, N)).astype(jnp.int32)], axis=-1
    )

    gx = grid[:, :DG, :WG, 0]
    gy = grid[:, :DG, :WG, 1]
    gz = grid[:, :DG, :WG, 2]

    idx = _project(gx, gy, gz, fparams, iparams).reshape(B, N, P)

    table = features.transpose(0, 1, 3, 4, 2).reshape(B * N * RPI, C)
    table = jnp.concatenate(
        [table, jnp.zeros((1, C), f32), jnp.full((1, C), -jnp.inf, f32)], axis=0
    )

    out_t = _gather_max()(table, idx)
    return out_t.reshape(B, C, HC, DG, WG).reshape(B, C * HC, DG, WG)


# speedup vs baseline: 25.9634x; 9.9092x over previous
import functools

import jax
import jax.numpy as jnp
from jax import lax
from jax.experimental import pallas as pl
from jax.experimental.pallas import tpu as pltpu
from jax.experimental.pallas import tpu_sc as plsc

B, N, C, H, W = 2, 6, 64, 112, 200
HC, DG, WG = 7, 200, 200
P = HC * DG * WG
RPI = H * W
ZROW = B * N * RPI
MROW = ZROW + 1
ZOFFS = [2.0, 1.5, 1.0, 0.5, 0.0, -0.5, -1.0]

NCORE, NSUB, LANES = 2, 16, 16
NW = NCORE * NSUB
LZROW = RPI
LMROW = RPI + 1
ROWS2 = RPI + 8
CPW = C // NW
K2 = 4000
NCH = P // K2
NPASS = N // 2


def _atan_pos(r):
    big = r > 2.414213562373095
    mid = r > 0.4142135623730951
    x = jnp.where(big, -1.0 / r, jnp.where(mid, (r - 1.0) / (r + 1.0), r))
    y = jnp.where(
        big,
        jnp.float32(1.5707963267948966),
        jnp.where(mid, jnp.float32(0.7853981633974483), jnp.float32(0.0)),
    )
    z = x * x
    p = (
        ((8.05374449538e-2 * z - 1.38776856032e-1) * z + 1.99777106478e-1) * z
        - 3.33329491539e-1
    ) * z * x + x
    return y + p


def _project_body(gx_ref, gy_ref, gz_ref, fp_ref, ip_ref, idx_ref):
    b = pl.program_id(0)
    i = pl.program_id(1)

    def fp(j):
        return fp_ref[b, i, j]

    c00, c01, c02, c03 = fp(0), fp(1), fp(2), fp(3)
    c10, c11, c12, c13 = fp(4), fp(5), fp(6), fp(7)
    c20, c21, c22, c23 = fp(8), fp(9), fp(10), fp(11)
    fx, fy, cx, cy = fp(12), fp(13), fp(14), fp(15)
    d0, d1, d2, d5 = fp(16), fp(17), fp(18), fp(19)
    k1, k2, k3, p1, p2 = fp(20), fp(21), fp(22), fp(23), fp(24)
    sel_fe = fp(25) > 0.5
    pr00, pr01, pr10, pr11 = fp(26), fp(27), fp(28), fp(29)
    pt0, pt1 = fp(30), fp(31)
    base = ip_ref[b, i, 0]
    dropped = ip_ref[b, i, 1] == 1

    gx = gx_ref[0]
    gy = gy_ref[0]
    gz0 = gz_ref[0]

    def b16(v):
        return v.astype(jnp.bfloat16).astype(jnp.float32)

    gxb = b16(gx)
    gyb = b16(gy)

    for h in range(HC):
        gz = gz0 + ZOFFS[h]
        gzb = b16(gz)
        hx = (b16(c00) * gxb + b16(c01) * gyb) + b16(c02) * gzb + c03
        hy = (b16(c10) * gxb + b16(c11) * gyb) + b16(c12) * gzb + c13
        hz = (b16(c20) * gxb + b16(c21) * gyb) + b16(c22) * gzb + c23
        f = (hz > 0.0).astype(jnp.float32)
        px = (hx * f) / hz
        py = (hy * f) / hz

        x = (px - cx) / fx
        y = (py - cy) / fy

        r = jnp.sqrt(x * x + y * y)
        th = _atan_pos(r)
        t2 = th * th
        t4 = t2 * t2
        rad = th * (1.0 + d0 * t2 + d1 * t4 + d2 * t4 * t2 + d5 * t4 * t4) / r
        fex = ((x * rad) * fx + cx) * f
        fey = ((y * rad) * fy + cy) * f

        r2 = x * x + y * y
        radial = 1.0 + k1 * r2 + k2 * r2 * r2 + k3 * r2 * r2 * r2
        phx = (x * radial + (2.0 * p1 * x * y + p2 * (r2 + 2.0 * x * x))) * fx + cx
        phy = (y * radial + (p1 * (r2 + 2.0 * y * y) + 2.0 * p2 * x * y)) * fy + cy
        phx = phx * f
        phy = phy * f

        xd = b16(jnp.where(sel_fe, fex, phx))
        yd = b16(jnp.where(sel_fe, fey, phy))

        u = (b16(pr00) * xd + b16(pr01) * yd) + pt0
        v = (b16(pr10) * xd + b16(pr11) * yd) + pt1

        bx = u.astype(jnp.int32)
        by = v.astype(jnp.int32)
        vis = (bx > 0) & (bx < W) & (by < H) & (by > 0)
        xx = jnp.clip(bx, 0, W - 1)
        yy = jnp.clip(by, 0, H - 1)
        row = yy * W + xx
        idx = jnp.where(vis, row, LZROW)
        idx = jnp.where(dropped, LMROW, idx)
        idx_ref[0, 0, h] = idx


_project = pl.pallas_call(
    _project_body,
    grid=(B, N),
    in_specs=[
        pl.BlockSpec((1, DG, WG), lambda b, i: (b, 0, 0)),
        pl.BlockSpec((1, DG, WG), lambda b, i: (b, 0, 0)),
        pl.BlockSpec((1, DG, WG), lambda b, i: (b, 0, 0)),
        pl.BlockSpec(memory_space=pltpu.SMEM),
        pl.BlockSpec(memory_space=pltpu.SMEM),
    ],
    out_specs=pl.BlockSpec((1, 1, HC, DG, WG), lambda b, i: (b, i, 0, 0, 0)),
    out_shape=jax.ShapeDtypeStruct((B, N, HC, DG, WG), jnp.int32),
)


def _gather_max_body(table_hbm, idx_hbm, out_hbm, tbl_v, idx_v, acc_v, prev_v):
    cc = lax.axis_index("s") * NCORE + lax.axis_index("c")

    for p_ in range(NPASS):
        i0 = 2 * p_
        for b in range(B):
            pltpu.sync_copy(table_hbm.at[b, i0, cc], tbl_v.at[0])
            pltpu.sync_copy(table_hbm.at[b, i0 + 1, cc], tbl_v.at[1])

            def chunk(t, carry, p_=p_, b=b):
                pb = t * K2
                pltpu.sync_copy(idx_hbm.at[b, i0, pl.ds(pb, K2)], idx_v.at[0])
                pltpu.sync_copy(idx_hbm.at[b, i0 + 1, pl.ds(pb, K2)], idx_v.at[1])
                if p_ > 0:
                    pltpu.sync_copy(out_hbm.at[b, cc, :, pl.ds(pb, K2)], prev_v)

                def vec(j, carry2):
                    s = pl.ds(j * LANES, LANES)
                    rv0 = idx_v[0, s]
                    rv1 = idx_v[1, s]
                    for ch in range(CPW):
                        chv = jnp.full((LANES,), ch, jnp.int32)
                        g0 = plsc.load_gather(tbl_v, [jnp.zeros((LANES,), jnp.int32), chv, rv0])
                        g1 = plsc.load_gather(tbl_v, [jnp.full((LANES,), 1, jnp.int32), chv, rv1])
                        m = jnp.maximum(g0, g1)
                        if p_ > 0:
                            m = jnp.maximum(m, prev_v[ch, s])
                        acc_v[ch, s] = m
                    return carry2

                lax.fori_loop(0, K2 // LANES, vec, 0)
                pltpu.sync_copy(acc_v, out_hbm.at[b, cc, :, pl.ds(pb, K2)])
                return carry

            lax.fori_loop(0, NCH, chunk, 0)


@functools.lru_cache(maxsize=1)
def _gather_max():
    return pl.kernel(
        _gather_max_body,
        out_type=jax.ShapeDtypeStruct((B, NW, CPW, P), jnp.float32),
        mesh=plsc.VectorSubcoreMesh(
            core_axis_name="c", subcore_axis_name="s",
            num_cores=NCORE, num_subcores=NSUB,
        ),
        scratch_types=[
            pltpu.VMEM((2, CPW, ROWS2), jnp.float32),
            pltpu.VMEM((2, K2), jnp.int32),
            pltpu.VMEM((CPW, K2), jnp.float32),
            pltpu.VMEM((CPW, K2), jnp.float32),
        ],
        compiler_params=pltpu.CompilerParams(
            use_tc_tiling_on_sc=False, needs_layout_passes=False
        ),
    )


@jax.jit
def kernel(features, ks, imu2cs, post_rots, post_trans, undists, grid, drop_idx, neck):
    f32 = jnp.float32
    calib = jnp.matmul(ks, imu2cs)

    fparams = jnp.concatenate(
        [
            calib.reshape(B, N, 12).astype(f32),
            jnp.stack(
                [ks[..., 0, 0], ks[..., 1, 1], ks[..., 0, 2], ks[..., 1, 2]], axis=-1
            ).astype(f32),
            jnp.stack(
                [undists[..., 0], undists[..., 1], undists[..., 2], undists[..., 5]],
                axis=-1,
            ).astype(f32),
            undists[..., 0:5].astype(f32),
            (undists[..., 6] == 1.0).astype(f32)[..., None],
            post_rots[..., :2, :2].reshape(B, N, 4).astype(f32),
            post_trans[..., :2].astype(f32),
        ],
        axis=-1,
    )

    base = ((jnp.arange(B, dtype=jnp.int32)[:, None] * N)
            + jnp.arange(N, dtype=jnp.int32)[None, :]) * RPI
    dropped = jnp.any(
        drop_idx[None, :] == jnp.arange(N, dtype=drop_idx.dtype)[:, None], axis=1
    )
    iparams = jnp.stack(
        [base, jnp.broadcast_to(dropped[None, :], (B, N)).astype(jnp.int32)], axis=-1
    )

    gx = grid[:, :DG, :WG, 0]
    gy = grid[:, :DG, :WG, 1]
    gz = grid[:, :DG, :WG, 2]

    idx = _project(gx, gy, gz, fparams, iparams).reshape(B, N, P)

    table = features.transpose(0, 1, 3, 4, 2)
    table = table.reshape(B, N, RPI, C)
    table = jnp.concatenate(
        [
            table,
            jnp.zeros((B, N, 1, C), f32),
            jnp.full((B, N, 1, C), -jnp.inf, f32),
            jnp.zeros((B, N, ROWS2 - RPI - 2, C), f32),
        ],
        axis=2,
    )
    table = table.reshape(B, N, ROWS2, NW, CPW).transpose(0, 1, 3, 4, 2)

    out_t = _gather_max()(table, idx)
    return out_t.reshape(B, C, HC, DG, WG).reshape(B, C * HC, DG, WG)


# speedup vs baseline: 27.8138x; 1.0713x over previous
import functools

import jax
import jax.numpy as jnp
from jax import lax
from jax.experimental import pallas as pl
from jax.experimental.pallas import tpu as pltpu
from jax.experimental.pallas import tpu_sc as plsc

B, N, C, H, W = 2, 6, 64, 112, 200
HC, DG, WG = 7, 200, 200
P = HC * DG * WG
RPI = H * W
ZROW = B * N * RPI
MROW = ZROW + 1
ZOFFS = [2.0, 1.5, 1.0, 0.5, 0.0, -0.5, -1.0]

NCORE, NSUB, LANES = 2, 16, 16
NW = NCORE * NSUB
LZROW = RPI
LMROW = RPI + 1
ROWS2 = RPI + 8
CPW = C // NW
K2 = 4000
NCH = P // K2
NPASS = N // 2


def _atan_pos(r):
    big = r > 2.414213562373095
    mid = r > 0.4142135623730951
    x = jnp.where(big, -1.0 / r, jnp.where(mid, (r - 1.0) / (r + 1.0), r))
    y = jnp.where(
        big,
        jnp.float32(1.5707963267948966),
        jnp.where(mid, jnp.float32(0.7853981633974483), jnp.float32(0.0)),
    )
    z = x * x
    p = (
        ((8.05374449538e-2 * z - 1.38776856032e-1) * z + 1.99777106478e-1) * z
        - 3.33329491539e-1
    ) * z * x + x
    return y + p


def _project_body(gx_ref, gy_ref, gz_ref, fp_ref, ip_ref, idx_ref):
    b = pl.program_id(0)
    i = pl.program_id(1)

    def fp(j):
        return fp_ref[b, i, j]

    c00, c01, c02, c03 = fp(0), fp(1), fp(2), fp(3)
    c10, c11, c12, c13 = fp(4), fp(5), fp(6), fp(7)
    c20, c21, c22, c23 = fp(8), fp(9), fp(10), fp(11)
    fx, fy, cx, cy = fp(12), fp(13), fp(14), fp(15)
    d0, d1, d2, d5 = fp(16), fp(17), fp(18), fp(19)
    k1, k2, k3, p1, p2 = fp(20), fp(21), fp(22), fp(23), fp(24)
    sel_fe = fp(25) > 0.5
    pr00, pr01, pr10, pr11 = fp(26), fp(27), fp(28), fp(29)
    pt0, pt1 = fp(30), fp(31)
    base = ip_ref[b, i, 0]
    dropped = ip_ref[b, i, 1] == 1

    gx = gx_ref[0]
    gy = gy_ref[0]
    gz0 = gz_ref[0]

    def b16(v):
        return v.astype(jnp.bfloat16).astype(jnp.float32)

    gxb = b16(gx)
    gyb = b16(gy)

    for h in range(HC):
        gz = gz0 + ZOFFS[h]
        gzb = b16(gz)
        hx = (b16(c00) * gxb + b16(c01) * gyb) + b16(c02) * gzb + c03
        hy = (b16(c10) * gxb + b16(c11) * gyb) + b16(c12) * gzb + c13
        hz = (b16(c20) * gxb + b16(c21) * gyb) + b16(c22) * gzb + c23
        f = (hz > 0.0).astype(jnp.float32)
        px = (hx * f) / hz
        py = (hy * f) / hz

        x = (px - cx) / fx
        y = (py - cy) / fy

        r = jnp.sqrt(x * x + y * y)
        th = _atan_pos(r)
        t2 = th * th
        t4 = t2 * t2
        rad = th * (1.0 + d0 * t2 + d1 * t4 + d2 * t4 * t2 + d5 * t4 * t4) / r
        fex = ((x * rad) * fx + cx) * f
        fey = ((y * rad) * fy + cy) * f

        r2 = x * x + y * y
        radial = 1.0 + k1 * r2 + k2 * r2 * r2 + k3 * r2 * r2 * r2
        phx = (x * radial + (2.0 * p1 * x * y + p2 * (r2 + 2.0 * x * x))) * fx + cx
        phy = (y * radial + (p1 * (r2 + 2.0 * y * y) + 2.0 * p2 * x * y)) * fy + cy
        phx = phx * f
        phy = phy * f

        xd = b16(jnp.where(sel_fe, fex, phx))
        yd = b16(jnp.where(sel_fe, fey, phy))

        u = (b16(pr00) * xd + b16(pr01) * yd) + pt0
        v = (b16(pr10) * xd + b16(pr11) * yd) + pt1

        bx = u.astype(jnp.int32)
        by = v.astype(jnp.int32)
        vis = (bx > 0) & (bx < W) & (by < H) & (by > 0)
        xx = jnp.clip(bx, 0, W - 1)
        yy = jnp.clip(by, 0, H - 1)
        row = yy * W + xx
        idx = jnp.where(vis, row, LZROW)
        idx = jnp.where(dropped, LMROW, idx)
        idx_ref[0, 0, h] = idx


_project = pl.pallas_call(
    _project_body,
    grid=(B, N),
    in_specs=[
        pl.BlockSpec((1, DG, WG), lambda b, i: (b, 0, 0)),
        pl.BlockSpec((1, DG, WG), lambda b, i: (b, 0, 0)),
        pl.BlockSpec((1, DG, WG), lambda b, i: (b, 0, 0)),
        pl.BlockSpec(memory_space=pltpu.SMEM),
        pl.BlockSpec(memory_space=pltpu.SMEM),
    ],
    out_specs=pl.BlockSpec((1, 1, HC, DG, WG), lambda b, i: (b, i, 0, 0, 0)),
    out_shape=jax.ShapeDtypeStruct((B, N, HC, DG, WG), jnp.int32),
)


def _gather_max_body(table_hbm, idx_hbm, out_hbm, tbl_v, idx_v, acc_v, prev_v):
    cc = lax.axis_index("s") * NCORE + lax.axis_index("c")
    neginf = jnp.full((LANES,), -jnp.inf, jnp.float32)
    zero = jnp.zeros((LANES,), jnp.float32)

    for p_ in range(NPASS):
        i0 = 2 * p_
        for b in range(B):
            pltpu.sync_copy(table_hbm.at[b, i0, pl.ds(cc * CPW, CPW)], tbl_v.at[0])
            pltpu.sync_copy(table_hbm.at[b, i0 + 1, pl.ds(cc * CPW, CPW)], tbl_v.at[1])

            def chunk(t, carry, p_=p_, b=b):
                pb = t * K2
                pltpu.sync_copy(idx_hbm.at[b, i0, pl.ds(pb, K2)], idx_v.at[0])
                pltpu.sync_copy(idx_hbm.at[b, i0 + 1, pl.ds(pb, K2)], idx_v.at[1])
                if p_ > 0:
                    pltpu.sync_copy(out_hbm.at[b, cc, :, pl.ds(pb, K2)], prev_v)

                def vec(j, carry2):
                    s = pl.ds(j * LANES, LANES)
                    rv0 = idx_v[0, s]
                    rv1 = idx_v[1, s]
                    in0 = rv0 < RPI
                    in1 = rv1 < RPI
                    sp0 = jnp.where(rv0 == LZROW, zero, neginf)
                    sp1 = jnp.where(rv1 == LZROW, zero, neginf)
                    rc0 = jnp.minimum(rv0, RPI - 1)
                    rc1 = jnp.minimum(rv1, RPI - 1)
                    zv = jnp.zeros((LANES,), jnp.int32)
                    ov = jnp.full((LANES,), 1, jnp.int32)
                    for ch in range(CPW):
                        chv = jnp.full((LANES,), ch, jnp.int32)
                        g0 = plsc.load_gather(tbl_v, [zv, chv, rc0])
                        g1 = plsc.load_gather(tbl_v, [ov, chv, rc1])
                        m = jnp.maximum(jnp.where(in0, g0, sp0), jnp.where(in1, g1, sp1))
                        if p_ > 0:
                            m = jnp.maximum(m, prev_v[ch, s])
                        acc_v[ch, s] = m
                    return carry2

                lax.fori_loop(0, K2 // LANES, vec, 0)
                pltpu.sync_copy(acc_v, out_hbm.at[b, cc, :, pl.ds(pb, K2)])
                return carry

            lax.fori_loop(0, NCH, chunk, 0)


@functools.lru_cache(maxsize=1)
def _gather_max():
    return pl.kernel(
        _gather_max_body,
        out_type=jax.ShapeDtypeStruct((B, NW, CPW, P), jnp.float32),
        mesh=plsc.VectorSubcoreMesh(
            core_axis_name="c", subcore_axis_name="s",
            num_cores=NCORE, num_subcores=NSUB,
        ),
        scratch_types=[
            pltpu.VMEM((2, CPW, RPI), jnp.float32),
            pltpu.VMEM((2, K2), jnp.int32),
            pltpu.VMEM((CPW, K2), jnp.float32),
            pltpu.VMEM((CPW, K2), jnp.float32),
        ],
        compiler_params=pltpu.CompilerParams(
            use_tc_tiling_on_sc=False, needs_layout_passes=False
        ),
    )


@jax.jit
def kernel(features, ks, imu2cs, post_rots, post_trans, undists, grid, drop_idx, neck):
    f32 = jnp.float32
    calib = jnp.matmul(ks, imu2cs)

    fparams = jnp.concatenate(
        [
            calib.reshape(B, N, 12).astype(f32),
            jnp.stack(
                [ks[..., 0, 0], ks[..., 1, 1], ks[..., 0, 2], ks[..., 1, 2]], axis=-1
            ).astype(f32),
            jnp.stack(
                [undists[..., 0], undists[..., 1], undists[..., 2], undists[..., 5]],
                axis=-1,
            ).astype(f32),
            undists[..., 0:5].astype(f32),
            (undists[..., 6] == 1.0).astype(f32)[..., None],
            post_rots[..., :2, :2].reshape(B, N, 4).astype(f32),
            post_trans[..., :2].astype(f32),
        ],
        axis=-1,
    )

    base = ((jnp.arange(B, dtype=jnp.int32)[:, None] * N)
            + jnp.arange(N, dtype=jnp.int32)[None, :]) * RPI
    dropped = jnp.any(
        drop_idx[None, :] == jnp.arange(N, dtype=drop_idx.dtype)[:, None], axis=1
    )
    iparams = jnp.stack(
        [base, jnp.broadcast_to(dropped[None, :], (B, N)).astype(jnp.int32)], axis=-1
    )

    gx = grid[:, :DG, :WG, 0]
    gy = grid[:, :DG, :WG, 1]
    gz = grid[:, :DG, :WG, 2]

    idx = _project(gx, gy, gz, fparams, iparams).reshape(B, N, P)

    table = features.reshape(B, N, C, RPI)

    out_t = _gather_max()(table, idx)
    return out_t.reshape(B, C, HC, DG, WG).reshape(B, C * HC, DG, WG)


# speedup vs baseline: 31.6101x; 1.1365x over previous
import functools

import jax
import jax.numpy as jnp
from jax import lax
from jax.experimental import pallas as pl
from jax.experimental.pallas import tpu as pltpu
from jax.experimental.pallas import tpu_sc as plsc

B, N, C, H, W = 2, 6, 64, 112, 200
HC, DG, WG = 7, 200, 200
P = HC * DG * WG
RPI = H * W
ZROW = B * N * RPI
MROW = ZROW + 1
ZOFFS = [2.0, 1.5, 1.0, 0.5, 0.0, -0.5, -1.0]

NCORE, NSUB, LANES = 2, 16, 16
NW = NCORE * NSUB
LZROW = RPI
LMROW = RPI + 1
ROWS2 = RPI + 8
CPW = C // NW
K2 = 4000
NCH = P // K2
NPASS = N // 2


def _atan_pos(r):
    big = r > 2.414213562373095
    mid = r > 0.4142135623730951
    x = jnp.where(big, -1.0 / r, jnp.where(mid, (r - 1.0) / (r + 1.0), r))
    y = jnp.where(
        big,
        jnp.float32(1.5707963267948966),
        jnp.where(mid, jnp.float32(0.7853981633974483), jnp.float32(0.0)),
    )
    z = x * x
    p = (
        ((8.05374449538e-2 * z - 1.38776856032e-1) * z + 1.99777106478e-1) * z
        - 3.33329491539e-1
    ) * z * x + x
    return y + p


def _project_body(gx_ref, gy_ref, gz_ref, fp_ref, ip_ref, idx_ref):
    b = pl.program_id(0)
    i = pl.program_id(1)

    def fp(j):
        return fp_ref[b, i, j]

    c00, c01, c02, c03 = fp(0), fp(1), fp(2), fp(3)
    c10, c11, c12, c13 = fp(4), fp(5), fp(6), fp(7)
    c20, c21, c22, c23 = fp(8), fp(9), fp(10), fp(11)
    fx, fy, cx, cy = fp(12), fp(13), fp(14), fp(15)
    d0, d1, d2, d5 = fp(16), fp(17), fp(18), fp(19)
    k1, k2, k3, p1, p2 = fp(20), fp(21), fp(22), fp(23), fp(24)
    sel_fe = fp(25) > 0.5
    pr00, pr01, pr10, pr11 = fp(26), fp(27), fp(28), fp(29)
    pt0, pt1 = fp(30), fp(31)
    base = ip_ref[b, i, 0]
    dropped = ip_ref[b, i, 1] == 1

    gx = gx_ref[0]
    gy = gy_ref[0]
    gz0 = gz_ref[0]

    def b16(v):
        return v.astype(jnp.bfloat16).astype(jnp.float32)

    gxb = b16(gx)
    gyb = b16(gy)

    for h in range(HC):
        gz = gz0 + ZOFFS[h]
        gzb = b16(gz)
        hx = (b16(c00) * gxb + b16(c01) * gyb) + b16(c02) * gzb + c03
        hy = (b16(c10) * gxb + b16(c11) * gyb) + b16(c12) * gzb + c13
        hz = (b16(c20) * gxb + b16(c21) * gyb) + b16(c22) * gzb + c23
        f = (hz > 0.0).astype(jnp.float32)
        px = (hx * f) / hz
        py = (hy * f) / hz

        x = (px - cx) / fx
        y = (py - cy) / fy

        r = jnp.sqrt(x * x + y * y)
        th = _atan_pos(r)
        t2 = th * th
        t4 = t2 * t2
        rad = th * (1.0 + d0 * t2 + d1 * t4 + d2 * t4 * t2 + d5 * t4 * t4) / r
        fex = ((x * rad) * fx + cx) * f
        fey = ((y * rad) * fy + cy) * f

        r2 = x * x + y * y
        radial = 1.0 + k1 * r2 + k2 * r2 * r2 + k3 * r2 * r2 * r2
        phx = (x * radial + (2.0 * p1 * x * y + p2 * (r2 + 2.0 * x * x))) * fx + cx
        phy = (y * radial + (p1 * (r2 + 2.0 * y * y) + 2.0 * p2 * x * y)) * fy + cy
        phx = phx * f
        phy = phy * f

        xd = b16(jnp.where(sel_fe, fex, phx))
        yd = b16(jnp.where(sel_fe, fey, phy))

        u = (b16(pr00) * xd + b16(pr01) * yd) + pt0
        v = (b16(pr10) * xd + b16(pr11) * yd) + pt1

        bx = u.astype(jnp.int32)
        by = v.astype(jnp.int32)
        vis = (bx > 0) & (bx < W) & (by < H) & (by > 0)
        xx = jnp.clip(bx, 0, W - 1)
        yy = jnp.clip(by, 0, H - 1)
        row = yy * W + xx
        idx = jnp.where(vis, row, LZROW)
        idx = jnp.where(dropped, LMROW, idx)
        idx_ref[0, 0, h] = idx


_project = pl.pallas_call(
    _project_body,
    grid=(B, N),
    in_specs=[
        pl.BlockSpec((1, DG, WG), lambda b, i: (b, 0, 0)),
        pl.BlockSpec((1, DG, WG), lambda b, i: (b, 0, 0)),
        pl.BlockSpec((1, DG, WG), lambda b, i: (b, 0, 0)),
        pl.BlockSpec(memory_space=pltpu.SMEM),
        pl.BlockSpec(memory_space=pltpu.SMEM),
    ],
    out_specs=pl.BlockSpec((1, 1, HC, DG, WG), lambda b, i: (b, i, 0, 0, 0)),
    out_shape=jax.ShapeDtypeStruct((B, N, HC, DG, WG), jnp.int32),
)


def _gather_max_body(table_hbm, idx_hbm, out_hbm, tbl_v, idx_v, acc_v, prev_v, dsem):
    cc = lax.axis_index("s") * NCORE + lax.axis_index("c")
    neginf = jnp.full((LANES,), -jnp.inf, jnp.float32)
    zero = jnp.zeros((LANES,), jnp.float32)

    for p_ in range(NPASS):
        i0 = 2 * p_
        for b in range(B):
            pltpu.sync_copy(table_hbm.at[b, i0, pl.ds(cc * CPW, CPW)], tbl_v.at[0])
            pltpu.sync_copy(table_hbm.at[b, i0 + 1, pl.ds(cc * CPW, CPW)], tbl_v.at[1])

            def chunk(t, carry, p_=p_, b=b):
                pb = t * K2
                d0 = pltpu.async_copy(idx_hbm.at[b, i0, pl.ds(pb, K2)], idx_v.at[0], dsem)
                d1 = pltpu.async_copy(idx_hbm.at[b, i0 + 1, pl.ds(pb, K2)], idx_v.at[1], dsem)
                if p_ > 0:
                    d2 = pltpu.async_copy(out_hbm.at[b, cc, :, pl.ds(pb, K2)], prev_v, dsem)
                d0.wait()
                d1.wait()
                if p_ > 0:
                    d2.wait()

                def vec(j, carry2):
                    s = pl.ds(j * LANES, LANES)
                    rv0 = idx_v[0, s]
                    rv1 = idx_v[1, s]
                    in0 = rv0 < RPI
                    in1 = rv1 < RPI
                    sp0 = jnp.where(rv0 == LZROW, zero, neginf)
                    sp1 = jnp.where(rv1 == LZROW, zero, neginf)
                    rc0 = jnp.minimum(rv0, RPI - 1)
                    rc1 = jnp.minimum(rv1, RPI - 1)
                    zv = jnp.zeros((LANES,), jnp.int32)
                    ov = jnp.full((LANES,), 1, jnp.int32)
                    for ch in range(CPW):
                        chv = jnp.full((LANES,), ch, jnp.int32)
                        g0 = plsc.load_gather(tbl_v, [zv, chv, rc0])
                        g1 = plsc.load_gather(tbl_v, [ov, chv, rc1])
                        m = jnp.maximum(jnp.where(in0, g0, sp0), jnp.where(in1, g1, sp1))
                        if p_ > 0:
                            m = jnp.maximum(m, prev_v[ch, s])
                        acc_v[ch, s] = m
                    return carry2

                lax.fori_loop(0, K2 // LANES, vec, 0)
                pltpu.sync_copy(acc_v, out_hbm.at[b, cc, :, pl.ds(pb, K2)])
                return carry

            lax.fori_loop(0, NCH, chunk, 0)


@functools.lru_cache(maxsize=1)
def _gather_max():
    return pl.kernel(
        _gather_max_body,
        out_type=jax.ShapeDtypeStruct((B, NW, CPW, P), jnp.float32),
        mesh=plsc.VectorSubcoreMesh(
            core_axis_name="c", subcore_axis_name="s",
            num_cores=NCORE, num_subcores=NSUB,
        ),
        scratch_types=[
            pltpu.VMEM((2, CPW, RPI), jnp.float32),
            pltpu.VMEM((2, K2), jnp.int32),
            pltpu.VMEM((CPW, K2), jnp.float32),
            pltpu.VMEM((CPW, K2), jnp.float32),
            pltpu.SemaphoreType.DMA,
        ],
        compiler_params=pltpu.CompilerParams(
            use_tc_tiling_on_sc=False, needs_layout_passes=False
        ),
    )


@jax.jit
def kernel(features, ks, imu2cs, post_rots, post_trans, undists, grid, drop_idx, neck):
    f32 = jnp.float32
    calib = jnp.matmul(ks, imu2cs)

    fparams = jnp.concatenate(
        [
            calib.reshape(B, N, 12).astype(f32),
            jnp.stack(
                [ks[..., 0, 0], ks[..., 1, 1], ks[..., 0, 2], ks[..., 1, 2]], axis=-1
            ).astype(f32),
            jnp.stack(
                [undists[..., 0], undists[..., 1], undists[..., 2], undists[..., 5]],
                axis=-1,
            ).astype(f32),
            undists[..., 0:5].astype(f32),
            (undists[..., 6] == 1.0).astype(f32)[..., None],
            post_rots[..., :2, :2].reshape(B, N, 4).astype(f32),
            post_trans[..., :2].astype(f32),
        ],
        axis=-1,
    )

    base = ((jnp.arange(B, dtype=jnp.int32)[:, None] * N)
            + jnp.arange(N, dtype=jnp.int32)[None, :]) * RPI
    dropped = jnp.any(
        drop_idx[None, :] == jnp.arange(N, dtype=drop_idx.dtype)[:, None], axis=1
    )
    iparams = jnp.stack(
        [base, jnp.broadcast_to(dropped[None, :], (B, N)).astype(jnp.int32)], axis=-1
    )

    gx = grid[:, :DG, :WG, 0]
    gy = grid[:, :DG, :WG, 1]
    gz = grid[:, :DG, :WG, 2]

    idx = _project(gx, gy, gz, fparams, iparams).reshape(B, N, P)

    table = features.reshape(B, N, C, RPI)

    out_t = _gather_max()(table, idx)
    return out_t.reshape(B, C, HC, DG, WG).reshape(B, C * HC, DG, WG)


# speedup vs baseline: 32.5744x; 1.0305x over previous
import functools

import jax
import jax.numpy as jnp
from jax import lax
from jax.experimental import pallas as pl
from jax.experimental.pallas import tpu as pltpu
from jax.experimental.pallas import tpu_sc as plsc

B, N, C, H, W = 2, 6, 64, 112, 200
HC, DG, WG = 7, 200, 200
P = HC * DG * WG
RPI = H * W
ZROW = B * N * RPI
MROW = ZROW + 1
ZOFFS = [2.0, 1.5, 1.0, 0.5, 0.0, -0.5, -1.0]

NCORE, NSUB, LANES = 2, 16, 16
NW = NCORE * NSUB
LZROW = RPI
LMROW = RPI + 1
ROWS2 = RPI + 8
CPW = C // NW
K2 = 5600
NCH = P // K2
NPASS = N // 2


def _atan_pos(r):
    big = r > 2.414213562373095
    mid = r > 0.4142135623730951
    x = jnp.where(big, -1.0 / r, jnp.where(mid, (r - 1.0) / (r + 1.0), r))
    y = jnp.where(
        big,
        jnp.float32(1.5707963267948966),
        jnp.where(mid, jnp.float32(0.7853981633974483), jnp.float32(0.0)),
    )
    z = x * x
    p = (
        ((8.05374449538e-2 * z - 1.38776856032e-1) * z + 1.99777106478e-1) * z
        - 3.33329491539e-1
    ) * z * x + x
    return y + p


def _project_body(gx_ref, gy_ref, gz_ref, fp_ref, ip_ref, idx_ref):
    b = pl.program_id(0)
    i = pl.program_id(1)

    def fp(j):
        return fp_ref[b, i, j]

    c00, c01, c02, c03 = fp(0), fp(1), fp(2), fp(3)
    c10, c11, c12, c13 = fp(4), fp(5), fp(6), fp(7)
    c20, c21, c22, c23 = fp(8), fp(9), fp(10), fp(11)
    fx, fy, cx, cy = fp(12), fp(13), fp(14), fp(15)
    d0, d1, d2, d5 = fp(16), fp(17), fp(18), fp(19)
    k1, k2, k3, p1, p2 = fp(20), fp(21), fp(22), fp(23), fp(24)
    sel_fe = fp(25) > 0.5
    pr00, pr01, pr10, pr11 = fp(26), fp(27), fp(28), fp(29)
    pt0, pt1 = fp(30), fp(31)
    base = ip_ref[b, i, 0]
    dropped = ip_ref[b, i, 1] == 1

    gx = gx_ref[0]
    gy = gy_ref[0]
    gz0 = gz_ref[0]

    def b16(v):
        return v.astype(jnp.bfloat16).astype(jnp.float32)

    gxb = b16(gx)
    gyb = b16(gy)

    for h in range(HC):
        gz = gz0 + ZOFFS[h]
        gzb = b16(gz)
        hx = (b16(c00) * gxb + b16(c01) * gyb) + b16(c02) * gzb + c03
        hy = (b16(c10) * gxb + b16(c11) * gyb) + b16(c12) * gzb + c13
        hz = (b16(c20) * gxb + b16(c21) * gyb) + b16(c22) * gzb + c23
        f = (hz > 0.0).astype(jnp.float32)
        px = (hx * f) / hz
        py = (hy * f) / hz

        x = (px - cx) / fx
        y = (py - cy) / fy

        r = jnp.sqrt(x * x + y * y)
        th = _atan_pos(r)
        t2 = th * th
        t4 = t2 * t2
        rad = th * (1.0 + d0 * t2 + d1 * t4 + d2 * t4 * t2 + d5 * t4 * t4) / r
        fex = ((x * rad) * fx + cx) * f
        fey = ((y * rad) * fy + cy) * f

        r2 = x * x + y * y
        radial = 1.0 + k1 * r2 + k2 * r2 * r2 + k3 * r2 * r2 * r2
        phx = (x * radial + (2.0 * p1 * x * y + p2 * (r2 + 2.0 * x * x))) * fx + cx
        phy = (y * radial + (p1 * (r2 + 2.0 * y * y) + 2.0 * p2 * x * y)) * fy + cy
        phx = phx * f
        phy = phy * f

        xd = b16(jnp.where(sel_fe, fex, phx))
        yd = b16(jnp.where(sel_fe, fey, phy))

        u = (b16(pr00) * xd + b16(pr01) * yd) + pt0
        v = (b16(pr10) * xd + b16(pr11) * yd) + pt1

        bx = u.astype(jnp.int32)
        by = v.astype(jnp.int32)
        vis = (bx > 0) & (bx < W) & (by < H) & (by > 0)
        xx = jnp.clip(bx, 0, W - 1)
        yy = jnp.clip(by, 0, H - 1)
        row = yy * W + xx
        idx = jnp.where(vis, row, LZROW)
        idx = jnp.where(dropped, LMROW, idx)
        idx_ref[0, 0, h] = idx


_project = pl.pallas_call(
    _project_body,
    grid=(B, N),
    in_specs=[
        pl.BlockSpec((1, DG, WG), lambda b, i: (b, 0, 0)),
        pl.BlockSpec((1, DG, WG), lambda b, i: (b, 0, 0)),
        pl.BlockSpec((1, DG, WG), lambda b, i: (b, 0, 0)),
        pl.BlockSpec(memory_space=pltpu.SMEM),
        pl.BlockSpec(memory_space=pltpu.SMEM),
    ],
    out_specs=pl.BlockSpec((1, 1, HC, DG, WG), lambda b, i: (b, i, 0, 0, 0)),
    out_shape=jax.ShapeDtypeStruct((B, N, HC, DG, WG), jnp.int32),
)


def _gather_max_body(table_hbm, idx_hbm, out_hbm, tbl_v, idx_v, acc_v, prev_v, dsem):
    cc = lax.axis_index("s") * NCORE + lax.axis_index("c")
    neginf = jnp.full((LANES,), -jnp.inf, jnp.float32)
    zero = jnp.zeros((LANES,), jnp.float32)

    for p_ in range(NPASS):
        i0 = 2 * p_
        for b in range(B):
            pltpu.sync_copy(table_hbm.at[b, i0, pl.ds(cc * CPW, CPW)], tbl_v.at[0])
            pltpu.sync_copy(table_hbm.at[b, i0 + 1, pl.ds(cc * CPW, CPW)], tbl_v.at[1])

            def chunk(t, carry, p_=p_, b=b):
                pb = t * K2
                d0 = pltpu.async_copy(idx_hbm.at[b, i0, pl.ds(pb, K2)], idx_v.at[0], dsem)
                d1 = pltpu.async_copy(idx_hbm.at[b, i0 + 1, pl.ds(pb, K2)], idx_v.at[1], dsem)
                if p_ > 0:
                    d2 = pltpu.async_copy(out_hbm.at[b, cc, :, pl.ds(pb, K2)], prev_v, dsem)
                d0.wait()
                d1.wait()
                if p_ > 0:
                    d2.wait()

                def vec(j, carry2):
                    s = pl.ds(j * LANES, LANES)
                    rv0 = idx_v[0, s]
                    rv1 = idx_v[1, s]
                    in0 = rv0 < RPI
                    in1 = rv1 < RPI
                    sp0 = jnp.where(rv0 == LZROW, zero, neginf)
                    sp1 = jnp.where(rv1 == LZROW, zero, neginf)
                    rc0 = jnp.minimum(rv0, RPI - 1)
                    rc1 = jnp.minimum(rv1, RPI - 1)
                    zv = jnp.zeros((LANES,), jnp.int32)
                    ov = jnp.full((LANES,), 1, jnp.int32)
                    for ch in range(CPW):
                        chv = jnp.full((LANES,), ch, jnp.int32)
                        g0 = plsc.load_gather(tbl_v, [zv, chv, rc0])
                        g1 = plsc.load_gather(tbl_v, [ov, chv, rc1])
                        m = jnp.maximum(jnp.where(in0, g0, sp0), jnp.where(in1, g1, sp1))
                        if p_ > 0:
                            m = jnp.maximum(m, prev_v[ch, s])
                        acc_v[ch, s] = m
                    return carry2

                lax.fori_loop(0, K2 // LANES, vec, 0)
                pltpu.sync_copy(acc_v, out_hbm.at[b, cc, :, pl.ds(pb, K2)])
                return carry

            lax.fori_loop(0, NCH, chunk, 0)


@functools.lru_cache(maxsize=1)
def _gather_max():
    return pl.kernel(
        _gather_max_body,
        out_type=jax.ShapeDtypeStruct((B, NW, CPW, P), jnp.float32),
        mesh=plsc.VectorSubcoreMesh(
            core_axis_name="c", subcore_axis_name="s",
            num_cores=NCORE, num_subcores=NSUB,
        ),
        scratch_types=[
            pltpu.VMEM((2, CPW, RPI), jnp.float32),
            pltpu.VMEM((2, K2), jnp.int32),
            pltpu.VMEM((CPW, K2), jnp.float32),
            pltpu.VMEM((CPW, K2), jnp.float32),
            pltpu.SemaphoreType.DMA,
        ],
        compiler_params=pltpu.CompilerParams(
            use_tc_tiling_on_sc=False, needs_layout_passes=False
        ),
    )


@jax.jit
def kernel(features, ks, imu2cs, post_rots, post_trans, undists, grid, drop_idx, neck):
    f32 = jnp.float32
    calib = jnp.matmul(ks, imu2cs)

    fparams = jnp.concatenate(
        [
            calib.reshape(B, N, 12).astype(f32),
            jnp.stack(
                [ks[..., 0, 0], ks[..., 1, 1], ks[..., 0, 2], ks[..., 1, 2]], axis=-1
            ).astype(f32),
            jnp.stack(
                [undists[..., 0], undists[..., 1], undists[..., 2], undists[..., 5]],
                axis=-1,
            ).astype(f32),
            undists[..., 0:5].astype(f32),
            (undists[..., 6] == 1.0).astype(f32)[..., None],
            post_rots[..., :2, :2].reshape(B, N, 4).astype(f32),
            post_trans[..., :2].astype(f32),
        ],
        axis=-1,
    )

    base = ((jnp.arange(B, dtype=jnp.int32)[:, None] * N)
            + jnp.arange(N, dtype=jnp.int32)[None, :]) * RPI
    dropped = jnp.any(
        drop_idx[None, :] == jnp.arange(N, dtype=drop_idx.dtype)[:, None], axis=1
    )
    iparams = jnp.stack(
        [base, jnp.broadcast_to(dropped[None, :], (B, N)).astype(jnp.int32)], axis=-1
    )

    gx = grid[:, :DG, :WG, 0]
    gy = grid[:, :DG, :WG, 1]
    gz = grid[:, :DG, :WG, 2]

    idx = _project(gx, gy, gz, fparams, iparams).reshape(B, N, P)

    table = features.reshape(B, N, C, RPI)

    out_t = _gather_max()(table, idx)
    return out_t.reshape(B, C, HC, DG, WG).reshape(B, C * HC, DG, WG)
